# baseline (speedup 1.0000x reference)
import functools

import jax
import jax.numpy as jnp
import numpy as np
from jax import lax
from jax.experimental import pallas as pl
from jax.experimental.pallas import tpu as pltpu
from jax.experimental.pallas import tpu_sc as plsc

D = 192
D3 = 3 * D
H = 8
DH = D // H
FF = 384
N = 50000
NSETS = 1400
SS = 36
NFLAT = NSETS * SS
NW = 32
NPAD = 51200
CH = NPAD // NW
CHG = 40
NCHG = CH // CHG
NBUF = 4
GSET = 4
RB = GSET * SS
NBLK = NSETS // GSET
ZROW = NFLAT
ATT_ROWS = (NBLK + 1) * RB
RBD = 400

_mesh = plsc.VectorSubcoreMesh(core_axis_name="c", subcore_axis_name="s")
_SC_PARAMS = pltpu.CompilerParams(needs_layout_passes=False,
                                  use_tc_tiling_on_sc=False)


def _wid():
    return lax.axis_index("s") * 2 + lax.axis_index("c")


def _lane_shift_up(x, lane):
    idx = jnp.minimum(lane + 1, 15).reshape(16, 1)
    dn = lax.GatherDimensionNumbers(
        offset_dims=(), collapsed_slice_dims=(0,), start_index_map=(0,))
    return lax.gather(x, idx, dn, (1,),
                      mode=lax.GatherScatterMode.PROMISE_IN_BOUNDS)


def _winner_partials(flatw):
    @functools.partial(
        pl.kernel,
        out_type=jax.ShapeDtypeStruct((4 * NW * NPAD,), jnp.int32),
        mesh=_mesh,
        compiler_params=_SC_PARAMS,
        scratch_types=[
            pltpu.VMEM((NPAD,), jnp.int32),
            pltpu.VMEM((CH,), jnp.int32),
        ],
    )
    def k(flatw_hbm, out_hbm, lastpos, idxv):
        wid = _wid()
        base = wid * CH
        lane = lax.iota(jnp.int32, 16)

        for l in range(4):
            def initb(j, c):
                lastpos[pl.ds(j * 16, 16)] = jnp.full((16,), -1, jnp.int32)
                return c
            lax.fori_loop(0, NPAD // 16, initb, 0)
            pltpu.sync_copy(flatw_hbm.at[pl.ds(l * NPAD + base, CH)], idxv)

            def scat(i, c):
                key = idxv[pl.ds(i * 16, 16)]
                key2 = plsc.bitcast((key << 4) | lane, jnp.uint32)
                pos = i * 16 + lane + base
                sk, sv = plsc.sort_key_val(key2, pos)
                svox = lax.shift_right_logical(plsc.bitcast(sk, jnp.int32), 4)
                nxt = _lane_shift_up(svox, lane)
                win = (svox != nxt) | (lane == 15)
                plsc.store_scatter(lastpos, [svox], sv, mask=win)
                return c
            lax.fori_loop(0, CH // 16, scat, 0)

            pltpu.sync_copy(lastpos,
                            out_hbm.at[pl.ds((l * NW + wid) * NPAD, NPAD)])

    return k(flatw)


def _merge_winners(parts):
    @functools.partial(
        pl.kernel,
        out_type=jax.ShapeDtypeStruct((4 * NPAD,), jnp.int32),
        mesh=_mesh,
        compiler_params=_SC_PARAMS,
        scratch_types=[
            pltpu.VMEM((NW * CH,), jnp.int32),
            pltpu.VMEM((CH,), jnp.int32),
        ],
    )
    def k(part_hbm, g_hbm, pbuf, gbuf):
        base = _wid() * CH
        lane = lax.iota(jnp.int32, 16)
        for l in range(4):
            for t in range(NW):
                pltpu.sync_copy(
                    part_hbm.at[pl.ds((l * NW + t) * NPAD + base, CH)],
                    pbuf.at[pl.ds(t * CH, CH)])

            def gbody(j, c):
                m = pbuf[pl.ds(j * 16, 16)]
                for t in range(1, NW):
                    m = jnp.maximum(m, pbuf[pl.ds(t * CH + j * 16, 16)])
                slot = j * 16 + lane + base
                gbuf[pl.ds(j * 16, 16)] = jnp.where(
                    (m < 0) | (slot >= N), ZROW, m)
                return c
            lax.fori_loop(0, CH // 16, gbody, 0)
            pltpu.sync_copy(gbuf, g_hbm.at[pl.ds(l * NPAD + base, CH)])

    return k(parts)


def _gather_qkv(qkv_a, qkv_b, flata):
    @functools.partial(
        pl.kernel,
        out_type=jax.ShapeDtypeStruct((NPAD, D3), jnp.float32),
        mesh=_mesh,
        compiler_params=_SC_PARAMS,
        scratch_types=(
            [pltpu.VMEM((CH,), jnp.int32)]
            + [pltpu.VMEM((CHG, D3), jnp.float32) for _ in range(NBUF)]
            + [pltpu.SemaphoreType.DMA] * (2 * NBUF)
        ),
    )
    def k(qa_hbm, qb_hbm, idx_hbm, out_hbm, idxv, *rest):
        base = _wid() * CH
        pltpu.sync_copy(idx_hbm.at[pl.ds(base, CH)], idxv)
        rb = rest[0:NBUF]
        gs = rest[NBUF:2 * NBUF]
        ss = rest[2 * NBUF:3 * NBUF]
        srcs = (qa_hbm, qb_hbm, qa_hbm, qb_hbm)

        def gstart(ch, b):
            cb = pl.multiple_of(ch * CHG, 8)
            pltpu.async_copy(srcs[b].at[idxv.at[pl.ds(cb, CHG)]], rb[b], gs[b])

        def gwait(b):
            pltpu.make_async_copy(
                srcs[b].at[pl.ds(0, CHG)], rb[b], gs[b]).wait()

        def sstart(ch, b):
            cb = pl.multiple_of(base + ch * CHG, 8)
            pltpu.async_copy(rb[b], out_hbm.at[pl.ds(cb, CHG)], ss[b])

        def swait(b):
            pltpu.make_async_copy(
                rb[b], out_hbm.at[pl.ds(0, CHG)], ss[b]).wait()

        for b in range(NBUF):
            gstart(b, b)

        def lbody(i, c):
            for b in range(NBUF):
                ch = i * NBUF + b
                gwait(b)
                sstart(ch, b)
                swait(b)
                gstart(ch + NBUF, b)
            return c
        lax.fori_loop(0, NCHG // NBUF - 1, lbody, 0)

        for b in range(NBUF):
            ch = NCHG - NBUF + b
            gwait(b)
            sstart(ch, b)
            swait(b)

    return k(qkv_a, qkv_b, flata)


def _scatter_back(att_a, att_b, g_all, l):
    @functools.partial(
        pl.kernel,
        out_type=jax.ShapeDtypeStruct((NPAD, D), jnp.float32),
        mesh=_mesh,
        compiler_params=_SC_PARAMS,
        scratch_types=(
            [pltpu.VMEM((CH,), jnp.int32)]
            + [pltpu.VMEM((CHG, D), jnp.float32) for _ in range(NBUF)]
            + [pltpu.SemaphoreType.DMA] * (2 * NBUF)
        ),
    )
    def k(att_hbm, att2_hbm, g_hbm, src2_hbm, gv, *rest):
        base = _wid() * CH
        pltpu.sync_copy(g_hbm.at[pl.ds(l * NPAD + base, CH)], gv)
        rb = rest[0:NBUF]
        gs = rest[NBUF:2 * NBUF]
        ss = rest[2 * NBUF:3 * NBUF]
        srcs = (att_hbm, att2_hbm, att_hbm, att2_hbm)

        def gstart(ch, b):
            cb = pl.multiple_of(ch * CHG, 8)
            pltpu.async_copy(srcs[b].at[gv.at[pl.ds(cb, CHG)]], rb[b], gs[b])

        def gwait(b):
            pltpu.make_async_copy(
                srcs[b].at[pl.ds(0, CHG)], rb[b], gs[b]).wait()

        def sstart(ch, b):
            cb = pl.multiple_of(base + ch * CHG, 8)
            pltpu.async_copy(rb[b], src2_hbm.at[pl.ds(cb, CHG)], ss[b])

        def swait(b):
            pltpu.make_async_copy(
                rb[b], src2_hbm.at[pl.ds(0, CHG)], ss[b]).wait()

        for b in range(NBUF):
            gstart(b, b)

        def lbody(i, c):
            for b in range(NBUF):
                ch = i * NBUF + b
                gwait(b)
                sstart(ch, b)
                swait(b)
                gstart(ch + NBUF, b)
            return c
        lax.fori_loop(0, NCHG // NBUF - 1, lbody, 0)

        for b in range(NBUF):
            ch = NCHG - NBUF + b
            gwait(b)
            sstart(ch, b)
            swait(b)

    return k(att_a, att_b, g_all)


def _qkv0_body(x_ref, p_ref, wqk_ref, wv_ref, bq_ref, qa_ref, qb_ref):
    x = x_ref[...]
    t = x + p_ref[...]
    qk = jnp.dot(t, wqk_ref[...], preferred_element_type=jnp.float32)
    v = jnp.dot(x, wv_ref[...], preferred_element_type=jnp.float32)
    qkv = jnp.concatenate([qk, v], axis=1) + bq_ref[0:1, :]
    qa_ref[...] = qkv
    qb_ref[...] = qkv


def _qkv0(pillar, pos0, wqk, wv, bqkv):
    blk = lambda i: (i, 0)
    zero = lambda i: (0, 0)
    return pl.pallas_call(
        _qkv0_body,
        grid=(N // RBD,),
        in_specs=[
            pl.BlockSpec((RBD, D), blk),
            pl.BlockSpec((RBD, D), blk),
            pl.BlockSpec((D, 2 * D), zero),
            pl.BlockSpec((D, D), zero),
            pl.BlockSpec((8, D3), zero),
        ],
        out_specs=[pl.BlockSpec((RBD, D3), blk)] * 2,
        out_shape=[jax.ShapeDtypeStruct((N, D3), jnp.float32)] * 2,
    )(pillar, pos0, wqk, wv, bqkv)


def _attn_body(qkv_ref, wo_ref, bo_ref, atta_ref, attb_ref):
    i = pl.program_id(0)

    @pl.when(i < NBLK)
    def _():
        rs = lax.broadcasted_iota(jnp.int32, (RB, RB), 0) // SS
        cs = lax.broadcasted_iota(jnp.int32, (RB, RB), 1) // SS
        badd = jnp.where(rs == cs, 0.0, -1e9)
        qkv = qkv_ref[...]
        q = qkv[:, :D] * np.float32(1.0 / np.sqrt(DH))
        kk = qkv[:, D:2 * D]
        v = qkv[:, 2 * D:]
        outs = []
        for h in range(H):
            qh = q[:, h * DH:(h + 1) * DH]
            kh = kk[:, h * DH:(h + 1) * DH]
            vh = v[:, h * DH:(h + 1) * DH]
            s = lax.dot_general(qh, kh, (((1,), (1,)), ((), ())),
                                preferred_element_type=jnp.float32) + badd
            m = jnp.max(s, axis=1, keepdims=True)
            e = jnp.exp(s - m)
            den = jnp.sum(e, axis=1, keepdims=True)
            o = lax.dot_general(e, vh, (((1,), (0,)), ((), ())),
                                preferred_element_type=jnp.float32)
            outs.append(o / den)
        o = jnp.concatenate(outs, axis=1)
        att = (jnp.dot(o, wo_ref[...], preferred_element_type=jnp.float32)
               + bo_ref[0:1, :])
        atta_ref[...] = att
        attb_ref[...] = att

    @pl.when(i == NBLK)
    def _():
        atta_ref[...] = jnp.zeros((RB, D), jnp.float32)
        attb_ref[...] = jnp.zeros((RB, D), jnp.float32)


def _attn(qkvg, wo, bo):
    blk = lambda i: (jnp.minimum(i, NBLK - 1), 0)
    zero = lambda i: (0, 0)
    return pl.pallas_call(
        _attn_body,
        grid=(NBLK + 1,),
        in_specs=[
            pl.BlockSpec((RB, D3), blk),
            pl.BlockSpec((D, D), zero),
            pl.BlockSpec((8, D), zero),
        ],
        out_specs=[pl.BlockSpec((RB, D), lambda i: (i, 0))] * 2,
        out_shape=[jax.ShapeDtypeStruct((ATT_ROWS, D), jnp.float32)] * 2,
    )(qkvg, wo, bo)


def _ln(t, g, b):
    m = jnp.mean(t, axis=1, keepdims=True)
    c = t - m
    var = jnp.mean(c * c, axis=1, keepdims=True)
    return c * lax.rsqrt(var + 1e-5) * g + b


def _ffn_body(has_outer, has_t, *refs):
    refs = list(refs)
    x_ref = refs.pop(0)
    s2_ref = refs.pop(0)
    r_ref = refs.pop(0) if has_outer else None
    if has_t:
        pn_ref = refs.pop(0)
        wqkn_ref = refs.pop(0)
        wvn_ref = refs.pop(0)
        bqn_ref = refs.pop(0)
    w1_ref, w2_ref, vp_ref = refs[:3]
    out_refs = refs[3:]
    vp = vp_ref[...]
    b1 = vp[0:1, :]
    b2 = vp[1:2, :D]
    g1 = vp[2:3, :D]
    be1 = vp[3:4, :D]
    g2 = vp[4:5, :D]
    be2 = vp[5:6, :D]
    h0 = x_ref[...] + s2_ref[...]
    x1 = _ln(h0, g1, be1)
    f = jnp.maximum(jnp.dot(x1, w1_ref[...],
                            preferred_element_type=jnp.float32) + b1, 0.0)
    f = jnp.dot(f, w2_ref[...], preferred_element_type=jnp.float32) + b2
    x2 = _ln(x1 + f, g2, be2)
    if has_outer:
        go = vp[6:7, :D]
        bo = vp[7:8, :D]
        x2 = _ln(r_ref[...] + x2, go, bo)
    out_refs[0][...] = x2
    if has_t:
        t = x2 + pn_ref[...]
        qk = jnp.dot(t, wqkn_ref[...], preferred_element_type=jnp.float32)
        v = jnp.dot(x2, wvn_ref[...], preferred_element_type=jnp.float32)
        qkv = jnp.concatenate([qk, v], axis=1) + bqn_ref[0:1, :]
        out_refs[1][...] = qkv
        out_refs[2][...] = qkv


def _ffn(x, src2, w1, w2, vpack, resid, nxt):
    grid = N // RBD
    blk = lambda i: (i, 0)
    zero = lambda i: (0, 0)
    has_outer = resid is not None
    has_t = nxt is not None
    ins = [x, src2]
    in_specs = [pl.BlockSpec((RBD, D), blk), pl.BlockSpec((RBD, D), blk)]
    if has_outer:
        ins.append(resid)
        in_specs.append(pl.BlockSpec((RBD, D), blk))
    if has_t:
        pos_next, wqkn, wvn, bqn = nxt
        ins += [pos_next, wqkn, wvn, bqn]
        in_specs += [
            pl.BlockSpec((RBD, D), blk),
            pl.BlockSpec((D, 2 * D), zero),
            pl.BlockSpec((D, D), zero),
            pl.BlockSpec((8, D3), zero),
        ]
    ins += [w1, w2, vpack]
    in_specs += [
        pl.BlockSpec((D, FF), zero),
        pl.BlockSpec((FF, D), zero),
        pl.BlockSpec((8, FF), zero),
    ]
    out_specs = [pl.BlockSpec((RBD, D), blk)]
    out_shape = [jax.ShapeDtypeStruct((N, D), jnp.float32)]
    if has_t:
        out_specs += [pl.BlockSpec((RBD, D3), blk)] * 2
        out_shape += [jax.ShapeDtypeStruct((N, D3), jnp.float32)] * 2
    out = pl.pallas_call(
        functools.partial(_ffn_body, has_outer, has_t),
        grid=(grid,),
        in_specs=in_specs,
        out_specs=out_specs,
        out_shape=out_shape,
    )(*ins)
    return out if has_t else (out[0], None, None)


def _pack_row(vec, width):
    return jnp.zeros((width,), jnp.float32).at[: vec.shape[0]].set(vec)


def kernel(pillar_features, pos_embed_tensor, params, outer_ln,
           set_voxel_inds_tensor_shift_0, set_voxel_inds_tensor_shift_1,
           set_voxel_masks_tensor_shift_0, set_voxel_masks_tensor_shift_1):
    del set_voxel_masks_tensor_shift_0, set_voxel_masks_tensor_shift_1
    inds = [set_voxel_inds_tensor_shift_0[0], set_voxel_inds_tensor_shift_0[1],
            set_voxel_inds_tensor_shift_1[0], set_voxel_inds_tensor_shift_1[1]]
    poss = [pos_embed_tensor[0, 0], pos_embed_tensor[0, 1],
            pos_embed_tensor[1, 0], pos_embed_tensor[1, 1]]
    flat = [i.reshape(-1).astype(jnp.int32) for i in inds]
    pad0 = jnp.zeros((NPAD - NFLAT,), jnp.int32)
    padn = jnp.full((NPAD - NFLAT,), N, jnp.int32)
    flata = [jnp.concatenate([f, pad0]) for f in flat]
    flatw = jnp.concatenate([jnp.concatenate([f, padn]) for f in flat])

    parts = _winner_partials(flatw)
    g_all = _merge_winners(parts)

    def wqk_of(p):
        return jnp.concatenate([p["Wq"], p["Wk"]], axis=1)

    def bqkv_of(p):
        return jnp.zeros((8, D3), jnp.float32).at[0].set(
            jnp.concatenate([p["bq"], p["bk"], p["bv"]]))

    x = pillar_features
    qkv_a, qkv_b = _qkv0(pillar_features, poss[0], wqk_of(params[0]),
                         params[0]["Wv"], bqkv_of(params[0]))
    res = x
    for l in range(4):
        p = params[l]
        bo8 = jnp.zeros((8, D), jnp.float32).at[0].set(p["bo"])
        has_outer = l % 2 == 1
        rows = [_pack_row(p["b1"], FF), _pack_row(p["b2"], FF),
                _pack_row(p["g1"], FF), _pack_row(p["be1"], FF),
                _pack_row(p["g2"], FF), _pack_row(p["be2"], FF)]
        if has_outer:
            ol = outer_ln[l // 2]
            rows += [_pack_row(ol["g"], FF), _pack_row(ol["b"], FF)]
        else:
            rows += [jnp.zeros((FF,), jnp.float32)] * 2
        vpack = jnp.stack(rows)

        qkvg = _gather_qkv(qkv_a, qkv_b, flata[l])
        att_a, att_b = _attn(qkvg, p["Wo"], bo8)
        src2 = _scatter_back(att_a, att_b, g_all, l)
        if l < 3:
            pn = params[l + 1]
            nxt = (poss[l + 1], wqk_of(pn), pn["Wv"], bqkv_of(pn))
        else:
            nxt = None
        x, qkv_a, qkv_b = _ffn(x, src2, p["W1"], p["W2"], vpack,
                               res if has_outer else None, nxt)
        if l == 1:
            res = x
    return x


# speedup vs baseline: 1.0342x; 1.0342x over previous
import functools

import jax
import jax.numpy as jnp
import numpy as np
from jax import lax
from jax.experimental import pallas as pl
from jax.experimental.pallas import tpu as pltpu
from jax.experimental.pallas import tpu_sc as plsc

D = 192
D3 = 3 * D
H = 8
DH = D // H
FF = 384
N = 50000
NSETS = 1400
SS = 36
NFLAT = NSETS * SS
NW = 32
NPAD = 51200
CH = NPAD // NW
CHG = 40
NCHG = CH // CHG
NBUF = 4
GSET = 4
RB = GSET * SS
NBLK = NSETS // GSET
ZROW = NFLAT
ATT_ROWS = (NBLK + 1) * RB
RBD = 1000

_mesh = plsc.VectorSubcoreMesh(core_axis_name="c", subcore_axis_name="s")
_SC_PARAMS = pltpu.CompilerParams(needs_layout_passes=False,
                                  use_tc_tiling_on_sc=False)


def _wid():
    return lax.axis_index("s") * 2 + lax.axis_index("c")


def _lane_shift_up(x, lane):
    idx = jnp.minimum(lane + 1, 15).reshape(16, 1)
    dn = lax.GatherDimensionNumbers(
        offset_dims=(), collapsed_slice_dims=(0,), start_index_map=(0,))
    return lax.gather(x, idx, dn, (1,),
                      mode=lax.GatherScatterMode.PROMISE_IN_BOUNDS)


def _winner_partials(flatw):
    @functools.partial(
        pl.kernel,
        out_type=jax.ShapeDtypeStruct((4 * NW * NPAD,), jnp.int32),
        mesh=_mesh,
        compiler_params=_SC_PARAMS,
        scratch_types=[
            pltpu.VMEM((NPAD,), jnp.int32),
            pltpu.VMEM((CH,), jnp.int32),
        ],
    )
    def k(flatw_hbm, out_hbm, lastpos, idxv):
        wid = _wid()
        base = wid * CH
        lane = lax.iota(jnp.int32, 16)

        for l in range(4):
            def initb(j, c):
                lastpos[pl.ds(j * 16, 16)] = jnp.full((16,), -1, jnp.int32)
                return c
            lax.fori_loop(0, NPAD // 16, initb, 0)
            pltpu.sync_copy(flatw_hbm.at[pl.ds(l * NPAD + base, CH)], idxv)

            def scat(i, c):
                key = idxv[pl.ds(i * 16, 16)]
                key2 = plsc.bitcast((key << 4) | lane, jnp.uint32)
                pos = i * 16 + lane + base
                sk, sv = plsc.sort_key_val(key2, pos)
                svox = lax.shift_right_logical(plsc.bitcast(sk, jnp.int32), 4)
                nxt = _lane_shift_up(svox, lane)
                win = (svox != nxt) | (lane == 15)
                plsc.store_scatter(lastpos, [svox], sv, mask=win)
                return c
            lax.fori_loop(0, CH // 16, scat, 0)

            pltpu.sync_copy(lastpos,
                            out_hbm.at[pl.ds((l * NW + wid) * NPAD, NPAD)])

    return k(flatw)


def _merge_winners(parts):
    @functools.partial(
        pl.kernel,
        out_type=jax.ShapeDtypeStruct((4 * NPAD,), jnp.int32),
        mesh=_mesh,
        compiler_params=_SC_PARAMS,
        scratch_types=[
            pltpu.VMEM((NW * CH,), jnp.int32),
            pltpu.VMEM((CH,), jnp.int32),
        ],
    )
    def k(part_hbm, g_hbm, pbuf, gbuf):
        base = _wid() * CH
        lane = lax.iota(jnp.int32, 16)
        for l in range(4):
            for t in range(NW):
                pltpu.sync_copy(
                    part_hbm.at[pl.ds((l * NW + t) * NPAD + base, CH)],
                    pbuf.at[pl.ds(t * CH, CH)])

            def gbody(j, c):
                m = pbuf[pl.ds(j * 16, 16)]
                for t in range(1, NW):
                    m = jnp.maximum(m, pbuf[pl.ds(t * CH + j * 16, 16)])
                slot = j * 16 + lane + base
                gbuf[pl.ds(j * 16, 16)] = jnp.where(
                    (m < 0) | (slot >= N), ZROW, m)
                return c
            lax.fori_loop(0, CH // 16, gbody, 0)
            pltpu.sync_copy(gbuf, g_hbm.at[pl.ds(l * NPAD + base, CH)])

    return k(parts)


def _gather_qkv(qkv_a, qkv_b, flata):
    @functools.partial(
        pl.kernel,
        out_type=jax.ShapeDtypeStruct((NPAD, D3), jnp.float32),
        mesh=_mesh,
        compiler_params=_SC_PARAMS,
        scratch_types=(
            [pltpu.VMEM((CH,), jnp.int32)]
            + [pltpu.VMEM((CHG, D3), jnp.float32) for _ in range(NBUF)]
            + [pltpu.SemaphoreType.DMA] * (2 * NBUF)
        ),
    )
    def k(qa_hbm, qb_hbm, idx_hbm, out_hbm, idxv, *rest):
        base = _wid() * CH
        pltpu.sync_copy(idx_hbm.at[pl.ds(base, CH)], idxv)
        rb = rest[0:NBUF]
        gs = rest[NBUF:2 * NBUF]
        ss = rest[2 * NBUF:3 * NBUF]
        srcs = (qa_hbm, qb_hbm, qa_hbm, qb_hbm)

        def gstart(ch, b):
            cb = pl.multiple_of(ch * CHG, 8)
            pltpu.async_copy(srcs[b].at[idxv.at[pl.ds(cb, CHG)]], rb[b], gs[b])

        def gwait(b):
            pltpu.make_async_copy(
                srcs[b].at[pl.ds(0, CHG)], rb[b], gs[b]).wait()

        def sstart(ch, b):
            cb = pl.multiple_of(base + ch * CHG, 8)
            pltpu.async_copy(rb[b], out_hbm.at[pl.ds(cb, CHG)], ss[b])

        def swait(b):
            pltpu.make_async_copy(
                rb[b], out_hbm.at[pl.ds(0, CHG)], ss[b]).wait()

        for b in range(NBUF):
            gstart(b, b)

        def lbody(i, c):
            for b in range(NBUF):
                ch = i * NBUF + b
                gwait(b)
                sstart(ch, b)
                swait(b)
                gstart(ch + NBUF, b)
            return c
        lax.fori_loop(0, NCHG // NBUF - 1, lbody, 0)

        for b in range(NBUF):
            ch = NCHG - NBUF + b
            gwait(b)
            sstart(ch, b)
            swait(b)

    return k(qkv_a, qkv_b, flata)


def _scatter_back(att_a, att_b, g_all, l):
    @functools.partial(
        pl.kernel,
        out_type=jax.ShapeDtypeStruct((NPAD, D), jnp.float32),
        mesh=_mesh,
        compiler_params=_SC_PARAMS,
        scratch_types=(
            [pltpu.VMEM((CH,), jnp.int32)]
            + [pltpu.VMEM((CHG, D), jnp.float32) for _ in range(NBUF)]
            + [pltpu.SemaphoreType.DMA] * (2 * NBUF)
        ),
    )
    def k(att_hbm, att2_hbm, g_hbm, src2_hbm, gv, *rest):
        base = _wid() * CH
        pltpu.sync_copy(g_hbm.at[pl.ds(l * NPAD + base, CH)], gv)
        rb = rest[0:NBUF]
        gs = rest[NBUF:2 * NBUF]
        ss = rest[2 * NBUF:3 * NBUF]
        srcs = (att_hbm, att2_hbm, att_hbm, att2_hbm)

        def gstart(ch, b):
            cb = pl.multiple_of(ch * CHG, 8)
            pltpu.async_copy(srcs[b].at[gv.at[pl.ds(cb, CHG)]], rb[b], gs[b])

        def gwait(b):
            pltpu.make_async_copy(
                srcs[b].at[pl.ds(0, CHG)], rb[b], gs[b]).wait()

        def sstart(ch, b):
            cb = pl.multiple_of(base + ch * CHG, 8)
            pltpu.async_copy(rb[b], src2_hbm.at[pl.ds(cb, CHG)], ss[b])

        def swait(b):
            pltpu.make_async_copy(
                rb[b], src2_hbm.at[pl.ds(0, CHG)], ss[b]).wait()

        for b in range(NBUF):
            gstart(b, b)

        def lbody(i, c):
            for b in range(NBUF):
                ch = i * NBUF + b
                gwait(b)
                sstart(ch, b)
                swait(b)
                gstart(ch + NBUF, b)
            return c
        lax.fori_loop(0, NCHG // NBUF - 1, lbody, 0)

        for b in range(NBUF):
            ch = NCHG - NBUF + b
            gwait(b)
            sstart(ch, b)
            swait(b)

    return k(att_a, att_b, g_all)


def _qkv0_body(x_ref, p_ref, wqk_ref, wv_ref, bq_ref, qa_ref, qb_ref):
    x = x_ref[...]
    t = x + p_ref[...]
    qk = jnp.dot(t, wqk_ref[...], preferred_element_type=jnp.float32)
    v = jnp.dot(x, wv_ref[...], preferred_element_type=jnp.float32)
    qkv = jnp.concatenate([qk, v], axis=1) + bq_ref[0:1, :]
    qa_ref[...] = qkv
    qb_ref[...] = qkv


def _qkv0(pillar, pos0, wqk, wv, bqkv):
    blk = lambda i: (i, 0)
    zero = lambda i: (0, 0)
    return pl.pallas_call(
        _qkv0_body,
        grid=(N // RBD,),
        in_specs=[
            pl.BlockSpec((RBD, D), blk),
            pl.BlockSpec((RBD, D), blk),
            pl.BlockSpec((D, 2 * D), zero),
            pl.BlockSpec((D, D), zero),
            pl.BlockSpec((8, D3), zero),
        ],
        out_specs=[pl.BlockSpec((RBD, D3), blk)] * 2,
        out_shape=[jax.ShapeDtypeStruct((N, D3), jnp.float32)] * 2,
    )(pillar, pos0, wqk, wv, bqkv)


def _attn_body(qkv_ref, wo_ref, bo_ref, atta_ref, attb_ref):
    i = pl.program_id(0)

    @pl.when(i < NBLK)
    def _():
        rs = lax.broadcasted_iota(jnp.int32, (RB, RB), 0) // SS
        cs = lax.broadcasted_iota(jnp.int32, (RB, RB), 1) // SS
        badd = jnp.where(rs == cs, 0.0, -1e9)
        qkv = qkv_ref[...]
        q = qkv[:, :D] * np.float32(1.0 / np.sqrt(DH))
        kk = qkv[:, D:2 * D]
        v = qkv[:, 2 * D:]
        outs = []
        for h in range(H):
            qh = q[:, h * DH:(h + 1) * DH]
            kh = kk[:, h * DH:(h + 1) * DH]
            vh = v[:, h * DH:(h + 1) * DH]
            s = lax.dot_general(qh, kh, (((1,), (1,)), ((), ())),
                                preferred_element_type=jnp.float32) + badd
            m = jnp.max(s, axis=1, keepdims=True)
            e = jnp.exp(s - m)
            den = jnp.sum(e, axis=1, keepdims=True)
            o = lax.dot_general(e, vh, (((1,), (0,)), ((), ())),
                                preferred_element_type=jnp.float32)
            outs.append(o / den)
        o = jnp.concatenate(outs, axis=1)
        att = (jnp.dot(o, wo_ref[...], preferred_element_type=jnp.float32)
               + bo_ref[0:1, :])
        atta_ref[...] = att
        attb_ref[...] = att

    @pl.when(i == NBLK)
    def _():
        atta_ref[...] = jnp.zeros((RB, D), jnp.float32)
        attb_ref[...] = jnp.zeros((RB, D), jnp.float32)


def _attn(qkvg, wo, bo):
    blk = lambda i: (jnp.minimum(i, NBLK - 1), 0)
    zero = lambda i: (0, 0)
    return pl.pallas_call(
        _attn_body,
        grid=(NBLK + 1,),
        in_specs=[
            pl.BlockSpec((RB, D3), blk),
            pl.BlockSpec((D, D), zero),
            pl.BlockSpec((8, D), zero),
        ],
        out_specs=[pl.BlockSpec((RB, D), lambda i: (i, 0))] * 2,
        out_shape=[jax.ShapeDtypeStruct((ATT_ROWS, D), jnp.float32)] * 2,
    )(qkvg, wo, bo)


def _ln(t, g, b):
    m = jnp.mean(t, axis=1, keepdims=True)
    c = t - m
    var = jnp.mean(c * c, axis=1, keepdims=True)
    return c * lax.rsqrt(var + 1e-5) * g + b


def _ffn_body(has_outer, has_t, *refs):
    refs = list(refs)
    x_ref = refs.pop(0)
    s2_ref = refs.pop(0)
    r_ref = refs.pop(0) if has_outer else None
    if has_t:
        pn_ref = refs.pop(0)
        wqkn_ref = refs.pop(0)
        wvn_ref = refs.pop(0)
        bqn_ref = refs.pop(0)
    w1_ref, w2_ref, vp_ref = refs[:3]
    out_refs = refs[3:]
    vp = vp_ref[...]
    b1 = vp[0:1, :]
    b2 = vp[1:2, :D]
    g1 = vp[2:3, :D]
    be1 = vp[3:4, :D]
    g2 = vp[4:5, :D]
    be2 = vp[5:6, :D]
    h0 = x_ref[...] + s2_ref[...]
    x1 = _ln(h0, g1, be1)
    f = jnp.maximum(jnp.dot(x1, w1_ref[...],
                            preferred_element_type=jnp.float32) + b1, 0.0)
    f = jnp.dot(f, w2_ref[...], preferred_element_type=jnp.float32) + b2
    x2 = _ln(x1 + f, g2, be2)
    if has_outer:
        go = vp[6:7, :D]
        bo = vp[7:8, :D]
        x2 = _ln(r_ref[...] + x2, go, bo)
    out_refs[0][...] = x2
    if has_t:
        t = x2 + pn_ref[...]
        qk = jnp.dot(t, wqkn_ref[...], preferred_element_type=jnp.float32)
        v = jnp.dot(x2, wvn_ref[...], preferred_element_type=jnp.float32)
        qkv = jnp.concatenate([qk, v], axis=1) + bqn_ref[0:1, :]
        out_refs[1][...] = qkv
        out_refs[2][...] = qkv


def _ffn(x, src2, w1, w2, vpack, resid, nxt):
    grid = N // RBD
    blk = lambda i: (i, 0)
    zero = lambda i: (0, 0)
    has_outer = resid is not None
    has_t = nxt is not None
    ins = [x, src2]
    in_specs = [pl.BlockSpec((RBD, D), blk), pl.BlockSpec((RBD, D), blk)]
    if has_outer:
        ins.append(resid)
        in_specs.append(pl.BlockSpec((RBD, D), blk))
    if has_t:
        pos_next, wqkn, wvn, bqn = nxt
        ins += [pos_next, wqkn, wvn, bqn]
        in_specs += [
            pl.BlockSpec((RBD, D), blk),
            pl.BlockSpec((D, 2 * D), zero),
            pl.BlockSpec((D, D), zero),
            pl.BlockSpec((8, D3), zero),
        ]
    ins += [w1, w2, vpack]
    in_specs += [
        pl.BlockSpec((D, FF), zero),
        pl.BlockSpec((FF, D), zero),
        pl.BlockSpec((8, FF), zero),
    ]
    out_specs = [pl.BlockSpec((RBD, D), blk)]
    out_shape = [jax.ShapeDtypeStruct((N, D), jnp.float32)]
    if has_t:
        out_specs += [pl.BlockSpec((RBD, D3), blk)] * 2
        out_shape += [jax.ShapeDtypeStruct((N, D3), jnp.float32)] * 2
    out = pl.pallas_call(
        functools.partial(_ffn_body, has_outer, has_t),
        grid=(grid,),
        in_specs=in_specs,
        out_specs=out_specs,
        out_shape=out_shape,
    )(*ins)
    return out if has_t else (out[0], None, None)


def _pack_row(vec, width):
    return jnp.zeros((width,), jnp.float32).at[: vec.shape[0]].set(vec)


def kernel(pillar_features, pos_embed_tensor, params, outer_ln,
           set_voxel_inds_tensor_shift_0, set_voxel_inds_tensor_shift_1,
           set_voxel_masks_tensor_shift_0, set_voxel_masks_tensor_shift_1):
    del set_voxel_masks_tensor_shift_0, set_voxel_masks_tensor_shift_1
    inds = [set_voxel_inds_tensor_shift_0[0], set_voxel_inds_tensor_shift_0[1],
            set_voxel_inds_tensor_shift_1[0], set_voxel_inds_tensor_shift_1[1]]
    poss = [pos_embed_tensor[0, 0], pos_embed_tensor[0, 1],
            pos_embed_tensor[1, 0], pos_embed_tensor[1, 1]]
    flat = [i.reshape(-1).astype(jnp.int32) for i in inds]
    pad0 = jnp.zeros((NPAD - NFLAT,), jnp.int32)
    padn = jnp.full((NPAD - NFLAT,), N, jnp.int32)
    flata = [jnp.concatenate([f, pad0]) for f in flat]
    flatw = jnp.concatenate([jnp.concatenate([f, padn]) for f in flat])

    parts = _winner_partials(flatw)
    g_all = _merge_winners(parts)

    def wqk_of(p):
        return jnp.concatenate([p["Wq"], p["Wk"]], axis=1)

    def bqkv_of(p):
        return jnp.zeros((8, D3), jnp.float32).at[0].set(
            jnp.concatenate([p["bq"], p["bk"], p["bv"]]))

    x = pillar_features
    qkv_a, qkv_b = _qkv0(pillar_features, poss[0], wqk_of(params[0]),
                         params[0]["Wv"], bqkv_of(params[0]))
    res = x
    for l in range(4):
        p = params[l]
        bo8 = jnp.zeros((8, D), jnp.float32).at[0].set(p["bo"])
        has_outer = l % 2 == 1
        rows = [_pack_row(p["b1"], FF), _pack_row(p["b2"], FF),
                _pack_row(p["g1"], FF), _pack_row(p["be1"], FF),
                _pack_row(p["g2"], FF), _pack_row(p["be2"], FF)]
        if has_outer:
            ol = outer_ln[l // 2]
            rows += [_pack_row(ol["g"], FF), _pack_row(ol["b"], FF)]
        else:
            rows += [jnp.zeros((FF,), jnp.float32)] * 2
        vpack = jnp.stack(rows)

        qkvg = _gather_qkv(qkv_a, qkv_b, flata[l])
        att_a, att_b = _attn(qkvg, p["Wo"], bo8)
        src2 = _scatter_back(att_a, att_b, g_all, l)
        if l < 3:
            pn = params[l + 1]
            nxt = (poss[l + 1], wqk_of(pn), pn["Wv"], bqkv_of(pn))
        else:
            nxt = None
        x, qkv_a, qkv_b = _ffn(x, src2, p["W1"], p["W2"], vpack,
                               res if has_outer else None, nxt)
        if l == 1:
            res = x
    return x


# speedup vs baseline: 1.0377x; 1.0034x over previous
import functools

import jax
import jax.numpy as jnp
import numpy as np
from jax import lax
from jax.experimental import pallas as pl
from jax.experimental.pallas import tpu as pltpu
from jax.experimental.pallas import tpu_sc as plsc

D = 192
D3 = 3 * D
H = 8
DH = D // H
FF = 384
N = 50000
NSETS = 1400
SS = 36
NFLAT = NSETS * SS
NW = 32
NPAD = 51200
CH = NPAD // NW
CHG = 40
NCHG = CH // CHG
NBUF = 4
GSET = 4
RB = GSET * SS
NBLK = NSETS // GSET
ZROW = NFLAT
ATT_ROWS = (NBLK + 1) * RB
RBD = 2000

_mesh = plsc.VectorSubcoreMesh(core_axis_name="c", subcore_axis_name="s")
_SC_PARAMS = pltpu.CompilerParams(needs_layout_passes=False,
                                  use_tc_tiling_on_sc=False)


def _wid():
    return lax.axis_index("s") * 2 + lax.axis_index("c")


def _lane_shift_up(x, lane):
    idx = jnp.minimum(lane + 1, 15).reshape(16, 1)
    dn = lax.GatherDimensionNumbers(
        offset_dims=(), collapsed_slice_dims=(0,), start_index_map=(0,))
    return lax.gather(x, idx, dn, (1,),
                      mode=lax.GatherScatterMode.PROMISE_IN_BOUNDS)


def _winner_partials(flatw):
    @functools.partial(
        pl.kernel,
        out_type=jax.ShapeDtypeStruct((4 * NW * NPAD,), jnp.int32),
        mesh=_mesh,
        compiler_params=_SC_PARAMS,
        scratch_types=[
            pltpu.VMEM((NPAD,), jnp.int32),
            pltpu.VMEM((CH,), jnp.int32),
        ],
    )
    def k(flatw_hbm, out_hbm, lastpos, idxv):
        wid = _wid()
        base = wid * CH
        lane = lax.iota(jnp.int32, 16)

        for l in range(4):
            def initb(j, c):
                lastpos[pl.ds(j * 16, 16)] = jnp.full((16,), -1, jnp.int32)
                return c
            lax.fori_loop(0, NPAD // 16, initb, 0)
            pltpu.sync_copy(flatw_hbm.at[pl.ds(l * NPAD + base, CH)], idxv)

            def scat(i, c):
                key = idxv[pl.ds(i * 16, 16)]
                key2 = plsc.bitcast((key << 4) | lane, jnp.uint32)
                pos = i * 16 + lane + base
                sk, sv = plsc.sort_key_val(key2, pos)
                svox = lax.shift_right_logical(plsc.bitcast(sk, jnp.int32), 4)
                nxt = _lane_shift_up(svox, lane)
                win = (svox != nxt) | (lane == 15)
                plsc.store_scatter(lastpos, [svox], sv, mask=win)
                return c
            lax.fori_loop(0, CH // 16, scat, 0)

            pltpu.sync_copy(lastpos,
                            out_hbm.at[pl.ds((l * NW + wid) * NPAD, NPAD)])

    return k(flatw)


def _merge_winners(parts):
    @functools.partial(
        pl.kernel,
        out_type=jax.ShapeDtypeStruct((4 * NPAD,), jnp.int32),
        mesh=_mesh,
        compiler_params=_SC_PARAMS,
        scratch_types=[
            pltpu.VMEM((NW * CH,), jnp.int32),
            pltpu.VMEM((CH,), jnp.int32),
        ],
    )
    def k(part_hbm, g_hbm, pbuf, gbuf):
        base = _wid() * CH
        lane = lax.iota(jnp.int32, 16)
        for l in range(4):
            for t in range(NW):
                pltpu.sync_copy(
                    part_hbm.at[pl.ds((l * NW + t) * NPAD + base, CH)],
                    pbuf.at[pl.ds(t * CH, CH)])

            def gbody(j, c):
                m = pbuf[pl.ds(j * 16, 16)]
                for t in range(1, NW):
                    m = jnp.maximum(m, pbuf[pl.ds(t * CH + j * 16, 16)])
                slot = j * 16 + lane + base
                gbuf[pl.ds(j * 16, 16)] = jnp.where(
                    (m < 0) | (slot >= N), ZROW, m)
                return c
            lax.fori_loop(0, CH // 16, gbody, 0)
            pltpu.sync_copy(gbuf, g_hbm.at[pl.ds(l * NPAD + base, CH)])

    return k(parts)


def _gather_qkv(qkv_a, qkv_b, flata):
    @functools.partial(
        pl.kernel,
        out_type=jax.ShapeDtypeStruct((NPAD, D3), jnp.float32),
        mesh=_mesh,
        compiler_params=_SC_PARAMS,
        scratch_types=(
            [pltpu.VMEM((CH,), jnp.int32)]
            + [pltpu.VMEM((CHG, D3), jnp.float32) for _ in range(NBUF)]
            + [pltpu.SemaphoreType.DMA] * (2 * NBUF)
        ),
    )
    def k(qa_hbm, qb_hbm, idx_hbm, out_hbm, idxv, *rest):
        base = _wid() * CH
        pltpu.sync_copy(idx_hbm.at[pl.ds(base, CH)], idxv)
        rb = rest[0:NBUF]
        gs = rest[NBUF:2 * NBUF]
        ss = rest[2 * NBUF:3 * NBUF]
        srcs = (qa_hbm, qb_hbm, qa_hbm, qb_hbm)

        def gstart(ch, b):
            cb = pl.multiple_of(ch * CHG, 8)
            pltpu.async_copy(srcs[b].at[idxv.at[pl.ds(cb, CHG)]], rb[b], gs[b])

        def gwait(b):
            pltpu.make_async_copy(
                srcs[b].at[pl.ds(0, CHG)], rb[b], gs[b]).wait()

        def sstart(ch, b):
            cb = pl.multiple_of(base + ch * CHG, 8)
            pltpu.async_copy(rb[b], out_hbm.at[pl.ds(cb, CHG)], ss[b])

        def swait(b):
            pltpu.make_async_copy(
                rb[b], out_hbm.at[pl.ds(0, CHG)], ss[b]).wait()

        for b in range(NBUF):
            gstart(b, b)

        def lbody(i, c):
            for b in range(NBUF):
                ch = i * NBUF + b
                gwait(b)
                sstart(ch, b)
                swait(b)
                gstart(ch + NBUF, b)
            return c
        lax.fori_loop(0, NCHG // NBUF - 1, lbody, 0)

        for b in range(NBUF):
            ch = NCHG - NBUF + b
            gwait(b)
            sstart(ch, b)
            swait(b)

    return k(qkv_a, qkv_b, flata)


def _scatter_back(att_a, att_b, g_all, l):
    @functools.partial(
        pl.kernel,
        out_type=jax.ShapeDtypeStruct((NPAD, D), jnp.float32),
        mesh=_mesh,
        compiler_params=_SC_PARAMS,
        scratch_types=(
            [pltpu.VMEM((CH,), jnp.int32)]
            + [pltpu.VMEM((CHG, D), jnp.float32) for _ in range(NBUF)]
            + [pltpu.SemaphoreType.DMA] * (2 * NBUF)
        ),
    )
    def k(att_hbm, att2_hbm, g_hbm, src2_hbm, gv, *rest):
        base = _wid() * CH
        pltpu.sync_copy(g_hbm.at[pl.ds(l * NPAD + base, CH)], gv)
        rb = rest[0:NBUF]
        gs = rest[NBUF:2 * NBUF]
        ss = rest[2 * NBUF:3 * NBUF]
        srcs = (att_hbm, att2_hbm, att_hbm, att2_hbm)

        def gstart(ch, b):
            cb = pl.multiple_of(ch * CHG, 8)
            pltpu.async_copy(srcs[b].at[gv.at[pl.ds(cb, CHG)]], rb[b], gs[b])

        def gwait(b):
            pltpu.make_async_copy(
                srcs[b].at[pl.ds(0, CHG)], rb[b], gs[b]).wait()

        def sstart(ch, b):
            cb = pl.multiple_of(base + ch * CHG, 8)
            pltpu.async_copy(rb[b], src2_hbm.at[pl.ds(cb, CHG)], ss[b])

        def swait(b):
            pltpu.make_async_copy(
                rb[b], src2_hbm.at[pl.ds(0, CHG)], ss[b]).wait()

        for b in range(NBUF):
            gstart(b, b)

        def lbody(i, c):
            for b in range(NBUF):
                ch = i * NBUF + b
                gwait(b)
                sstart(ch, b)
                swait(b)
                gstart(ch + NBUF, b)
            return c
        lax.fori_loop(0, NCHG // NBUF - 1, lbody, 0)

        for b in range(NBUF):
            ch = NCHG - NBUF + b
            gwait(b)
            sstart(ch, b)
            swait(b)

    return k(att_a, att_b, g_all)


def _qkv0_body(x_ref, p_ref, wqk_ref, wv_ref, bq_ref, qa_ref, qb_ref):
    x = x_ref[...]
    t = x + p_ref[...]
    qk = jnp.dot(t, wqk_ref[...], preferred_element_type=jnp.float32)
    v = jnp.dot(x, wv_ref[...], preferred_element_type=jnp.float32)
    qkv = jnp.concatenate([qk, v], axis=1) + bq_ref[0:1, :]
    qa_ref[...] = qkv
    qb_ref[...] = qkv


def _qkv0(pillar, pos0, wqk, wv, bqkv):
    blk = lambda i: (i, 0)
    zero = lambda i: (0, 0)
    return pl.pallas_call(
        _qkv0_body,
        grid=(N // RBD,),
        in_specs=[
            pl.BlockSpec((RBD, D), blk),
            pl.BlockSpec((RBD, D), blk),
            pl.BlockSpec((D, 2 * D), zero),
            pl.BlockSpec((D, D), zero),
            pl.BlockSpec((8, D3), zero),
        ],
        out_specs=[pl.BlockSpec((RBD, D3), blk)] * 2,
        out_shape=[jax.ShapeDtypeStruct((N, D3), jnp.float32)] * 2,
    )(pillar, pos0, wqk, wv, bqkv)


def _attn_body(qkv_ref, wo_ref, bo_ref, atta_ref, attb_ref):
    i = pl.program_id(0)

    @pl.when(i < NBLK)
    def _():
        rs = lax.broadcasted_iota(jnp.int32, (RB, RB), 0) // SS
        cs = lax.broadcasted_iota(jnp.int32, (RB, RB), 1) // SS
        badd = jnp.where(rs == cs, 0.0, -1e9)
        qkv = qkv_ref[...]
        q = qkv[:, :D] * np.float32(1.0 / np.sqrt(DH))
        kk = qkv[:, D:2 * D]
        v = qkv[:, 2 * D:]
        outs = []
        for h in range(H):
            qh = q[:, h * DH:(h + 1) * DH]
            kh = kk[:, h * DH:(h + 1) * DH]
            vh = v[:, h * DH:(h + 1) * DH]
            s = lax.dot_general(qh, kh, (((1,), (1,)), ((), ())),
                                preferred_element_type=jnp.float32) + badd
            m = jnp.max(s, axis=1, keepdims=True)
            e = jnp.exp(s - m)
            den = jnp.sum(e, axis=1, keepdims=True)
            o = lax.dot_general(e, vh, (((1,), (0,)), ((), ())),
                                preferred_element_type=jnp.float32)
            outs.append(o / den)
        o = jnp.concatenate(outs, axis=1)
        att = (jnp.dot(o, wo_ref[...], preferred_element_type=jnp.float32)
               + bo_ref[0:1, :])
        atta_ref[...] = att
        attb_ref[...] = att

    @pl.when(i == NBLK)
    def _():
        atta_ref[...] = jnp.zeros((RB, D), jnp.float32)
        attb_ref[...] = jnp.zeros((RB, D), jnp.float32)


def _attn(qkvg, wo, bo):
    blk = lambda i: (jnp.minimum(i, NBLK - 1), 0)
    zero = lambda i: (0, 0)
    return pl.pallas_call(
        _attn_body,
        grid=(NBLK + 1,),
        in_specs=[
            pl.BlockSpec((RB, D3), blk),
            pl.BlockSpec((D, D), zero),
            pl.BlockSpec((8, D), zero),
        ],
        out_specs=[pl.BlockSpec((RB, D), lambda i: (i, 0))] * 2,
        out_shape=[jax.ShapeDtypeStruct((ATT_ROWS, D), jnp.float32)] * 2,
    )(qkvg, wo, bo)


def _ln(t, g, b):
    m = jnp.mean(t, axis=1, keepdims=True)
    c = t - m
    var = jnp.mean(c * c, axis=1, keepdims=True)
    return c * lax.rsqrt(var + 1e-5) * g + b


def _ffn_body(has_outer, has_t, *refs):
    refs = list(refs)
    x_ref = refs.pop(0)
    s2_ref = refs.pop(0)
    r_ref = refs.pop(0) if has_outer else None
    if has_t:
        pn_ref = refs.pop(0)
        wqkn_ref = refs.pop(0)
        wvn_ref = refs.pop(0)
        bqn_ref = refs.pop(0)
    w1_ref, w2_ref, vp_ref = refs[:3]
    out_refs = refs[3:]
    vp = vp_ref[...]
    b1 = vp[0:1, :]
    b2 = vp[1:2, :D]
    g1 = vp[2:3, :D]
    be1 = vp[3:4, :D]
    g2 = vp[4:5, :D]
    be2 = vp[5:6, :D]
    h0 = x_ref[...] + s2_ref[...]
    x1 = _ln(h0, g1, be1)
    f = jnp.maximum(jnp.dot(x1, w1_ref[...],
                            preferred_element_type=jnp.float32) + b1, 0.0)
    f = jnp.dot(f, w2_ref[...], preferred_element_type=jnp.float32) + b2
    x2 = _ln(x1 + f, g2, be2)
    if has_outer:
        go = vp[6:7, :D]
        bo = vp[7:8, :D]
        x2 = _ln(r_ref[...] + x2, go, bo)
    out_refs[0][...] = x2
    if has_t:
        t = x2 + pn_ref[...]
        qk = jnp.dot(t, wqkn_ref[...], preferred_element_type=jnp.float32)
        v = jnp.dot(x2, wvn_ref[...], preferred_element_type=jnp.float32)
        qkv = jnp.concatenate([qk, v], axis=1) + bqn_ref[0:1, :]
        out_refs[1][...] = qkv
        out_refs[2][...] = qkv


def _ffn(x, src2, w1, w2, vpack, resid, nxt):
    grid = N // RBD
    blk = lambda i: (i, 0)
    zero = lambda i: (0, 0)
    has_outer = resid is not None
    has_t = nxt is not None
    ins = [x, src2]
    in_specs = [pl.BlockSpec((RBD, D), blk), pl.BlockSpec((RBD, D), blk)]
    if has_outer:
        ins.append(resid)
        in_specs.append(pl.BlockSpec((RBD, D), blk))
    if has_t:
        pos_next, wqkn, wvn, bqn = nxt
        ins += [pos_next, wqkn, wvn, bqn]
        in_specs += [
            pl.BlockSpec((RBD, D), blk),
            pl.BlockSpec((D, 2 * D), zero),
            pl.BlockSpec((D, D), zero),
            pl.BlockSpec((8, D3), zero),
        ]
    ins += [w1, w2, vpack]
    in_specs += [
        pl.BlockSpec((D, FF), zero),
        pl.BlockSpec((FF, D), zero),
        pl.BlockSpec((8, FF), zero),
    ]
    out_specs = [pl.BlockSpec((RBD, D), blk)]
    out_shape = [jax.ShapeDtypeStruct((N, D), jnp.float32)]
    if has_t:
        out_specs += [pl.BlockSpec((RBD, D3), blk)] * 2
        out_shape += [jax.ShapeDtypeStruct((N, D3), jnp.float32)] * 2
    out = pl.pallas_call(
        functools.partial(_ffn_body, has_outer, has_t),
        grid=(grid,),
        in_specs=in_specs,
        out_specs=out_specs,
        out_shape=out_shape,
    )(*ins)
    return out if has_t else (out[0], None, None)


def _pack_row(vec, width):
    return jnp.zeros((width,), jnp.float32).at[: vec.shape[0]].set(vec)


def kernel(pillar_features, pos_embed_tensor, params, outer_ln,
           set_voxel_inds_tensor_shift_0, set_voxel_inds_tensor_shift_1,
           set_voxel_masks_tensor_shift_0, set_voxel_masks_tensor_shift_1):
    del set_voxel_masks_tensor_shift_0, set_voxel_masks_tensor_shift_1
    inds = [set_voxel_inds_tensor_shift_0[0], set_voxel_inds_tensor_shift_0[1],
            set_voxel_inds_tensor_shift_1[0], set_voxel_inds_tensor_shift_1[1]]
    poss = [pos_embed_tensor[0, 0], pos_embed_tensor[0, 1],
            pos_embed_tensor[1, 0], pos_embed_tensor[1, 1]]
    flat = [i.reshape(-1).astype(jnp.int32) for i in inds]
    pad0 = jnp.zeros((NPAD - NFLAT,), jnp.int32)
    padn = jnp.full((NPAD - NFLAT,), N, jnp.int32)
    flata = [jnp.concatenate([f, pad0]) for f in flat]
    flatw = jnp.concatenate([jnp.concatenate([f, padn]) for f in flat])

    parts = _winner_partials(flatw)
    g_all = _merge_winners(parts)

    def wqk_of(p):
        return jnp.concatenate([p["Wq"], p["Wk"]], axis=1)

    def bqkv_of(p):
        return jnp.zeros((8, D3), jnp.float32).at[0].set(
            jnp.concatenate([p["bq"], p["bk"], p["bv"]]))

    x = pillar_features
    qkv_a, qkv_b = _qkv0(pillar_features, poss[0], wqk_of(params[0]),
                         params[0]["Wv"], bqkv_of(params[0]))
    res = x
    for l in range(4):
        p = params[l]
        bo8 = jnp.zeros((8, D), jnp.float32).at[0].set(p["bo"])
        has_outer = l % 2 == 1
        rows = [_pack_row(p["b1"], FF), _pack_row(p["b2"], FF),
                _pack_row(p["g1"], FF), _pack_row(p["be1"], FF),
                _pack_row(p["g2"], FF), _pack_row(p["be2"], FF)]
        if has_outer:
            ol = outer_ln[l // 2]
            rows += [_pack_row(ol["g"], FF), _pack_row(ol["b"], FF)]
        else:
            rows += [jnp.zeros((FF,), jnp.float32)] * 2
        vpack = jnp.stack(rows)

        qkvg = _gather_qkv(qkv_a, qkv_b, flata[l])
        att_a, att_b = _attn(qkvg, p["Wo"], bo8)
        src2 = _scatter_back(att_a, att_b, g_all, l)
        if l < 3:
            pn = params[l + 1]
            nxt = (poss[l + 1], wqk_of(pn), pn["Wv"], bqkv_of(pn))
        else:
            nxt = None
        x, qkv_a, qkv_b = _ffn(x, src2, p["W1"], p["W2"], vpack,
                               res if has_outer else None, nxt)
        if l == 1:
            res = x
    return x
